# baseline (speedup 1.0000x reference)
import functools

import jax
import jax.numpy as jnp
from jax import lax
from jax.experimental import pallas as pl
from jax.experimental.pallas import tpu as pltpu
from jax.experimental.pallas import tpu_sc as plsc

_B, _S, _H = 64, 512, 768
_EPS = 1e-12


def _make_sc_gather(V, H, N):
    info = plsc.get_sparse_core_info()
    NC, NS = info.num_cores, info.num_subcores
    NW = NC * NS
    TPW = N // NW
    CH = 32
    NCHUNK = TPW // CH
    DEPTH = 4
    mesh = plsc.VectorSubcoreMesh(core_axis_name="c", subcore_axis_name="s")

    @functools.partial(
        pl.kernel,
        mesh=mesh,
        out_type=jax.ShapeDtypeStruct((N, H), jnp.float32),
        scratch_types=(
            [pltpu.VMEM((TPW,), jnp.int32)]
            + [pltpu.VMEM((CH, H), jnp.float32)] * DEPTH
            + [pltpu.SemaphoreType.DMA] * (2 * DEPTH)
        ),
    )
    def sc_gather(table_hbm, ids_hbm, out_hbm, idx_v, *bufs_sems):
        bufs = bufs_sems[:DEPTH]
        gsems = bufs_sems[DEPTH:2 * DEPTH]
        ssems = bufs_sems[2 * DEPTH:]
        wid = lax.axis_index("s") * NC + lax.axis_index("c")
        base = wid * TPW
        pltpu.sync_copy(ids_hbm.at[pl.ds(base, TPW)], idx_v)

        def gather_desc(c):
            b = c % DEPTH
            return pltpu.make_async_copy(
                table_hbm.at[idx_v.at[pl.ds(c * CH, CH)]], bufs[b], gsems[b])

        def scatter_desc(c):
            b = c % DEPTH
            return pltpu.make_async_copy(
                bufs[b], out_hbm.at[pl.ds(base + c * CH, CH)], ssems[b])

        for c in range(DEPTH):
            gather_desc(c).start()
        for c in range(NCHUNK):
            if c >= DEPTH - 1 and c + 1 < NCHUNK:
                if c - (DEPTH - 1) >= 0:
                    scatter_desc(c - (DEPTH - 1)).wait()
                gather_desc(c + 1).start()
            gather_desc(c).wait()
            scatter_desc(c).start()
        for c in range(NCHUNK - DEPTH, NCHUNK):
            scatter_desc(c).wait()

    return sc_gather


def _tc_ln_body(g_ref, pos_ref, type_ref, ttf_ref, scale_ref, bias_ref, o_ref):
    t0 = type_ref[0:1, 0:1, :]
    t1 = type_ref[0:1, 1:2, :]
    x = g_ref[...] + pos_ref[...] + (t0 + ttf_ref[...] * (t1 - t0))
    mean = jnp.mean(x, axis=-1, keepdims=True)
    var = jnp.mean(x * x, axis=-1, keepdims=True) - mean * mean
    o_ref[...] = ((x - mean) * lax.rsqrt(var + _EPS)) * scale_ref[...] + bias_ref[...]


def _tc_ln_body_carry(g_ref, pos_ref, type_ref, ttf_ref, scale_ref, bias_ref,
                      carry_ref, o_ref):
    _tc_ln_body(g_ref, pos_ref, type_ref, ttf_ref, scale_ref, bias_ref, o_ref)


def _tc_ln_slab(gathered3, pos3, type3, ttf3, scale3, bias3, NB, B, boff,
                carry=None):
    BK, S, H = gathered3.shape
    ob = boff // NB
    in_specs = [
        pl.BlockSpec((NB, S, H), lambda g: (g, 0, 0)),
        pl.BlockSpec((1, S, H), lambda g: (0, 0, 0)),
        pl.BlockSpec((1, 2, H), lambda g: (0, 0, 0)),
        pl.BlockSpec((NB, S, 1), lambda g: (g, 0, 0)),
        pl.BlockSpec((1, 1, H), lambda g: (0, 0, 0)),
        pl.BlockSpec((1, 1, H), lambda g: (0, 0, 0)),
    ]
    args = [gathered3, pos3, type3, ttf3, scale3, bias3]
    body = _tc_ln_body
    aliases = {}
    if carry is not None:
        in_specs.append(pl.BlockSpec(memory_space=pltpu.MemorySpace.HBM))
        args.append(carry)
        body = _tc_ln_body_carry
        aliases = {6: 0}
    return pl.pallas_call(
        body,
        grid=(BK // NB,),
        in_specs=in_specs,
        out_specs=pl.BlockSpec((NB, S, H), lambda g: (g + ob, 0, 0)),
        out_shape=jax.ShapeDtypeStruct((B, S, H), jnp.float32),
        input_output_aliases=aliases,
        compiler_params=pltpu.CompilerParams(
            vmem_limit_bytes=100 * 1024 * 1024),
    )(*args)


def kernel(input_ids, token_type_ids, position_ids, attention_mask,
           word_emb, pos_emb, type_emb, ln_scale, ln_bias):
    B, S = input_ids.shape
    V, H = word_emb.shape
    N = B * S
    K = 1
    NB = 4
    BK = B // K
    NK = BK * S
    ids = input_ids.reshape(N).astype(jnp.int32)
    ttf = token_type_ids.reshape(B, S, 1).astype(jnp.float32)
    pos3 = pos_emb.reshape(1, S, H)
    type3 = type_emb.reshape(1, 2, H)
    scale3 = ln_scale.reshape(1, 1, H)
    bias3 = ln_bias.reshape(1, 1, H)

    sc_gather = _make_sc_gather(V, H, NK)
    slabs = [sc_gather(word_emb, ids[k * NK:(k + 1) * NK]) for k in range(K)]
    out = None
    for k in range(K):
        out = _tc_ln_slab(slabs[k].reshape(BK, S, H), pos3, type3,
                          ttf[k * BK:(k + 1) * BK], scale3, bias3,
                          NB, B, k * BK, carry=out)
    return out


# speedup vs baseline: 1.0017x; 1.0017x over previous
import functools

import jax
import jax.numpy as jnp
from jax import lax
from jax.experimental import pallas as pl
from jax.experimental.pallas import tpu as pltpu
from jax.experimental.pallas import tpu_sc as plsc

_B, _S, _H = 64, 512, 768
_EPS = 1e-12


def _make_sc_gather(V, H, N):
    info = plsc.get_sparse_core_info()
    NC, NS = info.num_cores, info.num_subcores
    NW = NC * NS
    TPW = N // NW
    CH = 64
    NCHUNK = TPW // CH
    mesh = plsc.VectorSubcoreMesh(core_axis_name="c", subcore_axis_name="s")

    @functools.partial(
        pl.kernel,
        mesh=mesh,
        out_type=jax.ShapeDtypeStruct((N, H), jnp.float32),
        scratch_types=[
            pltpu.VMEM((TPW,), jnp.int32),
            pltpu.VMEM((CH, H), jnp.float32),
            pltpu.VMEM((CH, H), jnp.float32),
            pltpu.SemaphoreType.DMA,
            pltpu.SemaphoreType.DMA,
            pltpu.SemaphoreType.DMA,
            pltpu.SemaphoreType.DMA,
        ],
    )
    def sc_gather(table_hbm, ids_hbm, out_hbm, idx_v, rows0, rows1,
                  gsem0, gsem1, ssem0, ssem1):
        wid = lax.axis_index("s") * NC + lax.axis_index("c")
        base = wid * TPW
        pltpu.sync_copy(ids_hbm.at[pl.ds(base, TPW)], idx_v)

        bufs = (rows0, rows1)
        gsems = (gsem0, gsem1)
        ssems = (ssem0, ssem1)

        gathers = [None, None]
        scatters = [None, None]
        gathers[0] = pltpu.async_copy(
            table_hbm.at[idx_v.at[pl.ds(0, CH)]], bufs[0], gsems[0])
        for c in range(NCHUNK):
            b = c % 2
            nb = (c + 1) % 2
            if c + 1 < NCHUNK:
                if scatters[nb] is not None:
                    scatters[nb].wait()
                gathers[nb] = pltpu.async_copy(
                    table_hbm.at[idx_v.at[pl.ds((c + 1) * CH, CH)]],
                    bufs[nb], gsems[nb])
            gathers[b].wait()
            scatters[b] = pltpu.async_copy(
                bufs[b], out_hbm.at[pl.ds(base + c * CH, CH)], ssems[b])
        scatters[0].wait()
        scatters[1].wait()

    return sc_gather


def _tc_ln_body(g_ref, pos_ref, type_ref, ttf_ref, scale_ref, bias_ref, o_ref):
    t0 = type_ref[0:1, 0:1, :]
    t1 = type_ref[0:1, 1:2, :]
    x = g_ref[...] + pos_ref[...] + (t0 + ttf_ref[...] * (t1 - t0))
    mean = jnp.mean(x, axis=-1, keepdims=True)
    var = jnp.mean(x * x, axis=-1, keepdims=True) - mean * mean
    o_ref[...] = ((x - mean) * lax.rsqrt(var + _EPS)) * scale_ref[...] + bias_ref[...]


def _tc_ln_body_carry(g_ref, pos_ref, type_ref, ttf_ref, scale_ref, bias_ref,
                      carry_ref, o_ref):
    _tc_ln_body(g_ref, pos_ref, type_ref, ttf_ref, scale_ref, bias_ref, o_ref)


def _tc_ln_slab(gathered3, pos3, type3, ttf3, scale3, bias3, NB, B, boff,
                carry=None):
    BK, S, H = gathered3.shape
    ob = boff // NB
    in_specs = [
        pl.BlockSpec((NB, S, H), lambda g: (g, 0, 0)),
        pl.BlockSpec((1, S, H), lambda g: (0, 0, 0)),
        pl.BlockSpec((1, 2, H), lambda g: (0, 0, 0)),
        pl.BlockSpec((NB, S, 1), lambda g: (g, 0, 0)),
        pl.BlockSpec((1, 1, H), lambda g: (0, 0, 0)),
        pl.BlockSpec((1, 1, H), lambda g: (0, 0, 0)),
    ]
    args = [gathered3, pos3, type3, ttf3, scale3, bias3]
    body = _tc_ln_body
    aliases = {}
    if carry is not None:
        in_specs.append(pl.BlockSpec(memory_space=pltpu.MemorySpace.HBM))
        args.append(carry)
        body = _tc_ln_body_carry
        aliases = {6: 0}
    return pl.pallas_call(
        body,
        grid=(BK // NB,),
        in_specs=in_specs,
        out_specs=pl.BlockSpec((NB, S, H), lambda g: (g + ob, 0, 0)),
        out_shape=jax.ShapeDtypeStruct((B, S, H), jnp.float32),
        input_output_aliases=aliases,
    )(*args)


def kernel(input_ids, token_type_ids, position_ids, attention_mask,
           word_emb, pos_emb, type_emb, ln_scale, ln_bias):
    B, S = input_ids.shape
    V, H = word_emb.shape
    N = B * S
    K = 1
    NB = 4
    BK = B // K
    NK = BK * S
    ids = input_ids.reshape(N).astype(jnp.int32)
    ttf = token_type_ids.reshape(B, S, 1).astype(jnp.float32)
    pos3 = pos_emb.reshape(1, S, H)
    type3 = type_emb.reshape(1, 2, H)
    scale3 = ln_scale.reshape(1, 1, H)
    bias3 = ln_bias.reshape(1, 1, H)

    sc_gather = _make_sc_gather(V, H, NK)
    slabs = [sc_gather(word_emb, ids[k * NK:(k + 1) * NK]) for k in range(K)]
    out = None
    for k in range(K):
        out = _tc_ln_slab(slabs[k].reshape(BK, S, H), pos3, type3,
                          ttf[k * BK:(k + 1) * BK], scale3, bias3,
                          NB, B, k * BK, carry=out)
    return out
